# baseline (speedup 1.0000x reference)
import functools

import jax
import jax.numpy as jnp
from jax import lax
from jax.experimental import pallas as pl
from jax.experimental.pallas import tpu as pltpu
from jax.experimental.pallas import tpu_sc as plsc

_BN_EPS = 1e-5

_N = 10000
_D = 128
_E = 320000
_NC = 2
_NS = 16
_NW = _NC * _NS
_NPAD = 10112
_RPS = _NPAD // _NS
_CHUNK = 112
_CPT0 = 144
_CPT1 = 40
_CPTT = _CPT0 + _CPT1
_EPAD = _NS * _CPTT * _CHUNK
_NBUF = 2

_mesh = plsc.VectorSubcoreMesh(core_axis_name="c", subcore_axis_name="s")


@functools.partial(
    pl.kernel,
    mesh=_mesh,
    out_type=jax.ShapeDtypeStruct((_NC, _NPAD, _D), jnp.float32),
    scratch_types=[
        pltpu.VMEM((_CPT0, _CHUNK), jnp.int32),
        pltpu.VMEM((_NBUF, _CHUNK), jnp.int32),
        pltpu.VMEM((_NBUF, _CHUNK), jnp.int32),
        pltpu.VMEM((_NBUF, _CHUNK, _D), jnp.float32),
        pltpu.VMEM_SHARED((_NPAD, _D), jnp.float32),
    ] + [pltpu.SemaphoreType.DMA] * _NBUF,
)
def _sc_aggregate(x_hbm, combo_hbm, out_hbm,
                  combo_v, sidx_v, didx_v, rows_v, agg_sh, *sems):
    cid = lax.axis_index("c")
    sid = lax.axis_index("s")
    nch = jnp.where(cid == 0, _CPT0, _CPT1)
    r0 = pl.multiple_of(sid * _RPS, 8)

    pltpu.sync_copy(x_hbm.at[pl.ds(r0, _RPS)], agg_sh.at[pl.ds(r0, _RPS)])

    @pl.when(cid == 0)
    def _():
        pltpu.sync_copy(combo_hbm.at[sid, pl.ds(0, _CPT0)], combo_v)

    @pl.when(cid == 1)
    def _():
        pltpu.sync_copy(combo_hbm.at[sid, pl.ds(_CPT0, _CPT1)],
                        combo_v.at[pl.ds(0, _CPT1)])

    plsc.subcore_barrier()

    def unpack_idx(j, p):
        for k in range(_CHUNK // 16):
            c = combo_v[j, pl.ds(k * 16, 16)]
            sidx_v[p, pl.ds(k * 16, 16)] = lax.shift_right_logical(c, 14)
            didx_v[p, pl.ds(k * 16, 16)] = lax.bitwise_and(c, 16383)

    for p in range(_NBUF):
        unpack_idx(p, p)
        pltpu.async_copy(x_hbm.at[sidx_v.at[p]], rows_v.at[p], sems[p])

    @pl.loop(0, nch, step=_NBUF)
    def _(j0):
        for p in range(_NBUF):
            j = j0 + p
            pltpu.make_async_copy(x_hbm.at[sidx_v.at[p]], rows_v.at[p],
                                  sems[p]).wait()
            pltpu.sync_copy(rows_v.at[p], agg_sh.at[didx_v.at[p]], add=True)

            @pl.when(j + _NBUF < nch)
            def _():
                unpack_idx(j + _NBUF, p)
                pltpu.async_copy(x_hbm.at[sidx_v.at[p]], rows_v.at[p],
                                 sems[p])

    plsc.subcore_barrier()
    pltpu.sync_copy(agg_sh.at[pl.ds(r0, _RPS)],
                    out_hbm.at[cid, pl.ds(r0, _RPS)])


def _tc_tail(x, parts, w1, b1, w2, b2, gamma, beta):
    def body(x_ref, p_ref, w1_ref, b1_ref, w2_ref, b2_ref, g_ref, bt_ref,
             o_ref):
        h = p_ref[0, :_N, :] + p_ref[1, :_N, :] - x_ref[...]
        h = jnp.dot(h, w1_ref[...], preferred_element_type=jnp.float32)
        h = jnp.maximum(h + b1_ref[...], 0.0)
        h = jnp.dot(h, w2_ref[...], preferred_element_type=jnp.float32)
        h = h + b2_ref[...]
        mean = jnp.mean(h, axis=0, keepdims=True)
        var = jnp.mean(h * h, axis=0, keepdims=True) - mean * mean
        scale = lax.rsqrt(var + _BN_EPS) * g_ref[...]
        o_ref[...] = jnp.maximum((h - mean) * scale + bt_ref[...], 0.0)

    return pl.pallas_call(
        body,
        out_shape=jax.ShapeDtypeStruct((_N, _D), jnp.float32),
    )(x, parts, w1, b1.reshape(1, _D), w2, b2.reshape(1, _D),
      gamma.reshape(1, _D), beta.reshape(1, _D))


def kernel(x, edge_index, W1, b1, W2, b2, gamma, beta):
    ei = edge_index.astype(jnp.int32)
    pad = _EPAD - _E
    combo = jnp.concatenate([
        jnp.left_shift(ei[0], 14) | ei[1],
        jnp.full((pad,), _N, jnp.int32),
    ]).reshape(_NS, _CPTT, _CHUNK)
    x_pad = jnp.concatenate([x, jnp.zeros((_NPAD - _N, _D), x.dtype)])
    parts = _sc_aggregate(x_pad, combo)
    return _tc_tail(x, parts, W1, b1, W2, b2, gamma, beta)


# speedup vs baseline: 1.4605x; 1.4605x over previous
import functools

import jax
import jax.numpy as jnp
from jax import lax
from jax.experimental import pallas as pl
from jax.experimental.pallas import tpu as pltpu
from jax.experimental.pallas import tpu_sc as plsc

_BN_EPS = 1e-5

_N = 10000
_D = 128
_HD = 64
_E = 320000
_NC = 2
_NS = 16
_NPAD = 10112
_RPS = _NPAD // _NS
_CHUNK = 128
_CPT = 160
_EPAD = _NS * _CPT * _CHUNK
_NBUF = 4

_mesh = plsc.VectorSubcoreMesh(core_axis_name="c", subcore_axis_name="s")


@functools.partial(
    pl.kernel,
    mesh=_mesh,
    compiler_params=pltpu.CompilerParams(use_tc_tiling_on_sc=False),
    out_type=jax.ShapeDtypeStruct((_NC, _NPAD, _HD), jnp.float32),
    scratch_types=[
        pltpu.VMEM((_CPT, _CHUNK), jnp.int32),
        pltpu.VMEM((_NBUF, _CHUNK), jnp.int32),
        pltpu.VMEM((_NBUF, _CHUNK), jnp.int32),
        pltpu.VMEM((_NBUF, _CHUNK, _HD), jnp.float32),
        pltpu.VMEM_SHARED((_NPAD, _HD), jnp.float32),
    ] + [pltpu.SemaphoreType.DMA] * _NBUF,
)
def _sc_aggregate(xh_hbm, combo_hbm, out_hbm,
                  combo_v, sidx_v, didx_v, rows_v, agg_sh, *sems):
    cid = lax.axis_index("c")
    sid = lax.axis_index("s")
    r0 = pl.multiple_of(sid * _RPS, 8)

    pltpu.sync_copy(xh_hbm.at[cid, pl.ds(r0, _RPS)],
                    agg_sh.at[pl.ds(r0, _RPS)])
    pltpu.sync_copy(combo_hbm.at[sid], combo_v)
    plsc.subcore_barrier()

    def unpack_idx(j, p):
        for k in range(_CHUNK // 16):
            c = combo_v[j, pl.ds(k * 16, 16)]
            sidx_v[p, pl.ds(k * 16, 16)] = lax.shift_right_logical(c, 14)
            didx_v[p, pl.ds(k * 16, 16)] = lax.bitwise_and(c, 16383)

    for p in range(_NBUF):
        unpack_idx(p, p)
        pltpu.async_copy(xh_hbm.at[cid].at[sidx_v.at[p]], rows_v.at[p],
                         sems[p])

    @pl.loop(0, _CPT, step=_NBUF)
    def _(j0):
        for p in range(_NBUF):
            j = j0 + p
            pltpu.make_async_copy(xh_hbm.at[cid].at[sidx_v.at[p]],
                                  rows_v.at[p], sems[p]).wait()
            pltpu.sync_copy(rows_v.at[p], agg_sh.at[didx_v.at[p]], add=True)

            @pl.when(j + _NBUF < _CPT)
            def _():
                unpack_idx(j + _NBUF, p)
                pltpu.async_copy(xh_hbm.at[cid].at[sidx_v.at[p]],
                                 rows_v.at[p], sems[p])

    plsc.subcore_barrier()
    pltpu.sync_copy(agg_sh.at[pl.ds(r0, _RPS)],
                    out_hbm.at[cid, pl.ds(r0, _RPS)])


def _tc_tail(parts, w1, b1, w2, b2, gamma, beta):
    def body(p_ref, w1_ref, b1_ref, w2_ref, b2_ref, g_ref, bt_ref, o_ref):
        h = jnp.concatenate([p_ref[0, :_N, :], p_ref[1, :_N, :]], axis=1)
        h = jnp.dot(h, w1_ref[...], preferred_element_type=jnp.float32)
        h = jnp.maximum(h + b1_ref[...], 0.0)
        h = jnp.dot(h, w2_ref[...], preferred_element_type=jnp.float32)
        h = h + b2_ref[...]
        mean = jnp.mean(h, axis=0, keepdims=True)
        var = jnp.mean(h * h, axis=0, keepdims=True) - mean * mean
        scale = lax.rsqrt(var + _BN_EPS) * g_ref[...]
        o_ref[...] = jnp.maximum((h - mean) * scale + bt_ref[...], 0.0)

    return pl.pallas_call(
        body,
        out_shape=jax.ShapeDtypeStruct((_N, _D), jnp.float32),
    )(parts, w1, b1.reshape(1, _D), w2, b2.reshape(1, _D),
      gamma.reshape(1, _D), beta.reshape(1, _D))


def kernel(x, edge_index, W1, b1, W2, b2, gamma, beta):
    ei = edge_index.astype(jnp.int32)
    pad = _EPAD - _E
    combo = jnp.concatenate([
        jnp.left_shift(ei[0], 14) | ei[1],
        jnp.full((pad,), _N, jnp.int32),
    ]).reshape(_NS, _CPT, _CHUNK)
    xh = jnp.stack([x[:, :_HD], x[:, _HD:]])
    xh = jnp.concatenate(
        [xh, jnp.zeros((_NC, _NPAD - _N, _HD), x.dtype)], axis=1)
    parts = _sc_aggregate(xh, combo)
    return _tc_tail(parts, W1, b1, W2, b2, gamma, beta)


# speedup vs baseline: 2.4186x; 1.6560x over previous
import functools

import jax
import jax.numpy as jnp
from jax import lax
from jax.experimental import pallas as pl
from jax.experimental.pallas import tpu as pltpu
from jax.experimental.pallas import tpu_sc as plsc

_BN_EPS = 1e-5

_N = 10000
_D = 128
_HD = 64
_E = 320000
_NC = 2
_NS = 16
_NPAD = 10112
_RPS = _NPAD // _NS
_CHUNK = 128
_CPT = 160
_EPAD = _NS * _CPT * _CHUNK
_NBUF = 2

_mesh = plsc.VectorSubcoreMesh(core_axis_name="c", subcore_axis_name="s")


@functools.partial(
    pl.kernel,
    mesh=_mesh,
    compiler_params=pltpu.CompilerParams(use_tc_tiling_on_sc=False),
    out_type=jax.ShapeDtypeStruct((_NC, _NPAD, _HD), jnp.float32),
    scratch_types=[
        pltpu.VMEM((_CPT // 2, _CHUNK), jnp.int32),
        pltpu.VMEM((_NBUF, _CHUNK), jnp.int32),
        pltpu.VMEM((_NBUF, _CHUNK), jnp.int32),
        pltpu.VMEM((_NBUF, _CHUNK, _HD), jnp.float32),
        pltpu.VMEM_SHARED((_NPAD, _HD), jnp.float32),
        pltpu.VMEM_SHARED((_NPAD, _HD), jnp.float32),
    ] + [pltpu.SemaphoreType.DMA] * _NBUF,
)
def _sc_aggregate(xh_hbm, combo_hbm, out_hbm,
                  combo_v, sidx_v, didx_v, rows_v, x_sh, agg_sh, *sems):
    cid = lax.axis_index("c")
    sid = lax.axis_index("s")
    r0 = pl.multiple_of(sid * _RPS, 8)

    pltpu.sync_copy(xh_hbm.at[cid, pl.ds(r0, _RPS)], x_sh.at[pl.ds(r0, _RPS)])
    pltpu.sync_copy(xh_hbm.at[cid, pl.ds(r0, _RPS)],
                    agg_sh.at[pl.ds(r0, _RPS)])
    plsc.subcore_barrier()

    def unpack_idx(j, p):
        for k in range(_CHUNK // 16):
            c = combo_v[j, pl.ds(k * 16, 16)]
            sidx_v[p, pl.ds(k * 16, 16)] = lax.shift_right_logical(c, 14)
            didx_v[p, pl.ds(k * 16, 16)] = lax.bitwise_and(c, 16383)

    half = _CPT // 2
    for h in range(2):
        pltpu.sync_copy(combo_hbm.at[sid, pl.ds(h * half, half)], combo_v)
        for p in range(_NBUF):
            unpack_idx(p, p)
            pltpu.async_copy(x_sh.at[sidx_v.at[p]], rows_v.at[p], sems[p])

        @pl.loop(0, half, step=_NBUF)
        def _(j0):
            for p in range(_NBUF):
                j = j0 + p
                pltpu.make_async_copy(x_sh.at[sidx_v.at[p]], rows_v.at[p],
                                      sems[p]).wait()
                pltpu.sync_copy(rows_v.at[p], agg_sh.at[didx_v.at[p]],
                                add=True)

                @pl.when(j + _NBUF < half)
                def _():
                    unpack_idx(j + _NBUF, p)
                    pltpu.async_copy(x_sh.at[sidx_v.at[p]], rows_v.at[p],
                                     sems[p])

    plsc.subcore_barrier()
    pltpu.sync_copy(agg_sh.at[pl.ds(r0, _RPS)],
                    out_hbm.at[cid, pl.ds(r0, _RPS)])


def _tc_tail(parts, w1, b1, w2, b2, gamma, beta):
    def body(p_ref, w1_ref, b1_ref, w2_ref, b2_ref, g_ref, bt_ref, o_ref):
        h = jnp.concatenate([p_ref[0, :_N, :], p_ref[1, :_N, :]], axis=1)
        h = jnp.dot(h, w1_ref[...], preferred_element_type=jnp.float32)
        h = jnp.maximum(h + b1_ref[...], 0.0)
        h = jnp.dot(h, w2_ref[...], preferred_element_type=jnp.float32)
        h = h + b2_ref[...]
        mean = jnp.mean(h, axis=0, keepdims=True)
        var = jnp.mean(h * h, axis=0, keepdims=True) - mean * mean
        scale = lax.rsqrt(var + _BN_EPS) * g_ref[...]
        o_ref[...] = jnp.maximum((h - mean) * scale + bt_ref[...], 0.0)

    return pl.pallas_call(
        body,
        out_shape=jax.ShapeDtypeStruct((_N, _D), jnp.float32),
    )(parts, w1, b1.reshape(1, _D), w2, b2.reshape(1, _D),
      gamma.reshape(1, _D), beta.reshape(1, _D))


def kernel(x, edge_index, W1, b1, W2, b2, gamma, beta):
    ei = edge_index.astype(jnp.int32)
    pad = _EPAD - _E
    combo = jnp.concatenate([
        jnp.left_shift(ei[0], 14) | ei[1],
        jnp.full((pad,), _N, jnp.int32),
    ]).reshape(_NS, _CPT, _CHUNK)
    xh = jnp.stack([x[:, :_HD], x[:, _HD:]])
    xh = jnp.concatenate(
        [xh, jnp.zeros((_NC, _NPAD - _N, _HD), x.dtype)], axis=1)
    parts = _sc_aggregate(xh, combo)
    return _tc_tail(parts, W1, b1, W2, b2, gamma, beta)


# speedup vs baseline: 2.4378x; 1.0079x over previous
import functools

import jax
import jax.numpy as jnp
from jax import lax
from jax.experimental import pallas as pl
from jax.experimental.pallas import tpu as pltpu
from jax.experimental.pallas import tpu_sc as plsc

_BN_EPS = 1e-5

_N = 10000
_D = 128
_HD = 64
_E = 320000
_NC = 2
_NS = 16
_NPAD = 10112
_RPS = _NPAD // _NS
_CHUNK = 128
_CPT = 160
_EPAD = _NS * _CPT * _CHUNK
_NBUF = 2

_mesh = plsc.VectorSubcoreMesh(core_axis_name="c", subcore_axis_name="s")


@functools.partial(
    pl.kernel,
    mesh=_mesh,
    compiler_params=pltpu.CompilerParams(use_tc_tiling_on_sc=False),
    out_type=jax.ShapeDtypeStruct((_NC, _NPAD, _HD), jnp.float32),
    scratch_types=[
        pltpu.VMEM((_CPT // 2, _CHUNK), jnp.int32),
        pltpu.VMEM((_NBUF, _CHUNK), jnp.int32),
        pltpu.VMEM((_NBUF, _CHUNK), jnp.int32),
        pltpu.VMEM((_NBUF, _CHUNK, _HD), jnp.float32),
        pltpu.VMEM_SHARED((_NPAD, _HD), jnp.float32),
        pltpu.VMEM_SHARED((_NPAD, _HD), jnp.float32),
    ] + [pltpu.SemaphoreType.DMA] * _NBUF,
)
def _sc_aggregate(xh_hbm, combo_hbm, out_hbm,
                  combo_v, sidx_v, didx_v, rows_v, x_sh, agg_sh, *sems):
    cid = lax.axis_index("c")
    sid = lax.axis_index("s")
    r0 = pl.multiple_of(sid * _RPS, 8)

    pltpu.sync_copy(xh_hbm.at[cid, pl.ds(r0, _RPS)], x_sh.at[pl.ds(r0, _RPS)])
    pltpu.sync_copy(xh_hbm.at[cid, pl.ds(r0, _RPS)],
                    agg_sh.at[pl.ds(r0, _RPS)])
    plsc.subcore_barrier()

    def unpack_idx(j, p):
        for k in range(_CHUNK // 16):
            c = combo_v[j, pl.ds(k * 16, 16)]
            sidx_v[p, pl.ds(k * 16, 16)] = lax.shift_right_logical(c, 14)
            didx_v[p, pl.ds(k * 16, 16)] = lax.bitwise_and(c, 16383)

    half = _CPT // 2
    for h in range(2):
        pltpu.sync_copy(combo_hbm.at[sid, pl.ds(h * half, half)], combo_v)
        for p in range(_NBUF):
            unpack_idx(p, p)
            pltpu.async_copy(x_sh.at[sidx_v.at[p]], rows_v.at[p], sems[p])

        @pl.loop(0, half, step=_NBUF)
        def _(j0):
            for p in range(_NBUF):
                j = j0 + p
                pltpu.make_async_copy(x_sh.at[sidx_v.at[p]], rows_v.at[p],
                                      sems[p]).wait()
                pltpu.sync_copy(rows_v.at[p], agg_sh.at[didx_v.at[p]],
                                add=True)

                @pl.when(j + _NBUF < half)
                def _():
                    unpack_idx(j + _NBUF, p)
                    pltpu.async_copy(x_sh.at[sidx_v.at[p]], rows_v.at[p],
                                     sems[p])

    plsc.subcore_barrier()
    pltpu.sync_copy(agg_sh.at[pl.ds(r0, _RPS)],
                    out_hbm.at[cid, pl.ds(r0, _RPS)])


def _tc_pack(srcdst):
    def body(x_ref, o_ref):
        o_ref[...] = jnp.left_shift(x_ref[0], 14) | x_ref[1]

    nrow = _EPAD // 128
    blk = nrow // 8
    return pl.pallas_call(
        body,
        grid=(8,),
        in_specs=[pl.BlockSpec((2, blk, 128), lambda i: (0, i, 0))],
        out_specs=pl.BlockSpec((blk, 128), lambda i: (i, 0)),
        out_shape=jax.ShapeDtypeStruct((nrow, 128), jnp.int32),
    )(srcdst)


def _tc_tail(parts, w1, b1, w2, b2, gamma, beta):
    def body(p_ref, w1_ref, b1_ref, w2_ref, b2_ref, g_ref, bt_ref, o_ref):
        h = jnp.concatenate([p_ref[0, :_N, :], p_ref[1, :_N, :]], axis=1)
        h = jnp.dot(h, w1_ref[...], preferred_element_type=jnp.float32)
        h = jnp.maximum(h + b1_ref[...], 0.0)
        h = jnp.dot(h, w2_ref[...], preferred_element_type=jnp.float32)
        h = h + b2_ref[...]
        mean = jnp.mean(h, axis=0, keepdims=True)
        var = jnp.mean(h * h, axis=0, keepdims=True) - mean * mean
        scale = lax.rsqrt(var + _BN_EPS) * g_ref[...]
        o_ref[...] = jnp.maximum((h - mean) * scale + bt_ref[...], 0.0)

    return pl.pallas_call(
        body,
        out_shape=jax.ShapeDtypeStruct((_N, _D), jnp.float32),
    )(parts, w1, b1.reshape(1, _D), w2, b2.reshape(1, _D),
      gamma.reshape(1, _D), beta.reshape(1, _D))


def kernel(x, edge_index, W1, b1, W2, b2, gamma, beta):
    ei = edge_index.astype(jnp.int32)
    pad = _EPAD - _E
    padcol = jnp.broadcast_to(jnp.array([[0], [_N]], jnp.int32), (2, pad))
    srcdst = jnp.concatenate([ei, padcol], axis=1)
    combo = _tc_pack(srcdst.reshape(2, _EPAD // 128, 128))
    combo = combo.reshape(_NS, _CPT, _CHUNK)
    xh = jnp.stack([x[:, :_HD], x[:, _HD:]])
    xh = jnp.concatenate(
        [xh, jnp.zeros((_NC, _NPAD - _N, _HD), x.dtype)], axis=1)
    parts = _sc_aggregate(xh, combo)
    return _tc_tail(parts, W1, b1, W2, b2, gamma, beta)


# speedup vs baseline: 2.7796x; 1.1402x over previous
import functools

import jax
import jax.numpy as jnp
from jax import lax
from jax.experimental import pallas as pl
from jax.experimental.pallas import tpu as pltpu
from jax.experimental.pallas import tpu_sc as plsc

_BN_EPS = 1e-5

_N = 10000
_D = 128
_HD = 64
_E = 320000
_NC = 2
_NS = 16
_NPAD = 10112
_RPS = _NPAD // _NS
_CHUNK = 128
_CPT = 160
_EPAD = _NS * _CPT * _CHUNK
_NBUF = 2

_mesh = plsc.VectorSubcoreMesh(core_axis_name="c", subcore_axis_name="s")


@functools.partial(
    pl.kernel,
    mesh=_mesh,
    compiler_params=pltpu.CompilerParams(use_tc_tiling_on_sc=False),
    out_type=jax.ShapeDtypeStruct((_NPAD, _D), jnp.float32),
    scratch_types=[
        pltpu.VMEM((_CPT // 2, _CHUNK), jnp.int32),
        pltpu.VMEM((_NBUF, _CHUNK), jnp.int32),
        pltpu.VMEM((_NBUF, _CHUNK), jnp.int32),
        pltpu.VMEM((_NBUF, _CHUNK, _HD), jnp.float32),
        pltpu.VMEM_SHARED((_NPAD, _HD), jnp.float32),
        pltpu.VMEM_SHARED((_NPAD, _HD), jnp.float32),
    ] + [pltpu.SemaphoreType.DMA] * _NBUF,
)
def _sc_aggregate(x_hbm, combo_hbm, out_hbm,
                  combo_v, sidx_v, didx_v, rows_v, x_sh, agg_sh, *sems):
    cid = lax.axis_index("c")
    sid = lax.axis_index("s")
    r0 = pl.multiple_of(sid * _RPS, 8)
    c0 = pl.multiple_of(cid * _HD, _HD)

    pltpu.sync_copy(x_hbm.at[pl.ds(r0, _RPS), pl.ds(c0, _HD)],
                    x_sh.at[pl.ds(r0, _RPS)])
    pltpu.sync_copy(x_hbm.at[pl.ds(r0, _RPS), pl.ds(c0, _HD)],
                    agg_sh.at[pl.ds(r0, _RPS)])
    plsc.subcore_barrier()

    def unpack_idx(j, p):
        for k in range(_CHUNK // 16):
            c = combo_v[j, pl.ds(k * 16, 16)]
            sidx_v[p, pl.ds(k * 16, 16)] = lax.shift_right_logical(c, 14)
            didx_v[p, pl.ds(k * 16, 16)] = lax.bitwise_and(c, 16383)

    half = _CPT // 2
    for h in range(2):
        pltpu.sync_copy(combo_hbm.at[sid, pl.ds(h * half, half)], combo_v)
        for p in range(_NBUF):
            unpack_idx(p, p)
            pltpu.async_copy(x_sh.at[sidx_v.at[p]], rows_v.at[p], sems[p])

        @pl.loop(0, half, step=_NBUF)
        def _(j0):
            for p in range(_NBUF):
                j = j0 + p
                pltpu.make_async_copy(x_sh.at[sidx_v.at[p]], rows_v.at[p],
                                      sems[p]).wait()
                pltpu.sync_copy(rows_v.at[p], agg_sh.at[didx_v.at[p]],
                                add=True)

                @pl.when(j + _NBUF < half)
                def _():
                    unpack_idx(j + _NBUF, p)
                    pltpu.async_copy(x_sh.at[sidx_v.at[p]], rows_v.at[p],
                                     sems[p])

    plsc.subcore_barrier()
    pltpu.sync_copy(agg_sh.at[pl.ds(r0, _RPS)],
                    out_hbm.at[pl.ds(r0, _RPS), pl.ds(c0, _HD)])


_EROW = _E // 128
_OROW = _EPAD // 128
_PBLK = _OROW // 8


def _tc_pack(srcdst):
    def body(x_ref, o_ref):
        i = pl.program_id(0)
        row = i * _PBLK + lax.broadcasted_iota(jnp.int32, (_PBLK, 128), 0)
        packed = jnp.left_shift(x_ref[0], 14) | x_ref[1]
        o_ref[...] = jnp.where(row < _EROW, packed, _N)

    return pl.pallas_call(
        body,
        grid=(8,),
        in_specs=[pl.BlockSpec((2, _PBLK, 128), lambda i: (0, i, 0))],
        out_specs=pl.BlockSpec((_PBLK, 128), lambda i: (i, 0)),
        out_shape=jax.ShapeDtypeStruct((_OROW, 128), jnp.int32),
    )(srcdst)


def _tc_tail(parts, w1, b1, w2, b2, gamma, beta):
    def body(p_ref, w1_ref, b1_ref, w2_ref, b2_ref, g_ref, bt_ref, o_ref):
        h = p_ref[:_N, :]
        h = jnp.dot(h, w1_ref[...], preferred_element_type=jnp.float32)
        h = jnp.maximum(h + b1_ref[...], 0.0)
        h = jnp.dot(h, w2_ref[...], preferred_element_type=jnp.float32)
        h = h + b2_ref[...]
        mean = jnp.mean(h, axis=0, keepdims=True)
        var = jnp.mean(h * h, axis=0, keepdims=True) - mean * mean
        scale = lax.rsqrt(var + _BN_EPS) * g_ref[...]
        o_ref[...] = jnp.maximum((h - mean) * scale + bt_ref[...], 0.0)

    return pl.pallas_call(
        body,
        out_shape=jax.ShapeDtypeStruct((_N, _D), jnp.float32),
    )(parts, w1, b1.reshape(1, _D), w2, b2.reshape(1, _D),
      gamma.reshape(1, _D), beta.reshape(1, _D))


def kernel(x, edge_index, W1, b1, W2, b2, gamma, beta):
    ei = edge_index.astype(jnp.int32)
    ei = jnp.pad(ei, ((0, 0), (0, _EPAD - _E)))
    combo = _tc_pack(ei.reshape(2, _OROW, 128))
    combo = combo.reshape(_NS, _CPT, _CHUNK)
    x_pad = jnp.concatenate([x, jnp.zeros((_NPAD - _N, _D), x.dtype)])
    parts = _sc_aggregate(x_pad, combo)
    return _tc_tail(parts, W1, b1, W2, b2, gamma, beta)


# speedup vs baseline: 2.8764x; 1.0348x over previous
import functools

import jax
import jax.numpy as jnp
from jax import lax
from jax.experimental import pallas as pl
from jax.experimental.pallas import tpu as pltpu
from jax.experimental.pallas import tpu_sc as plsc

_BN_EPS = 1e-5

_N = 10000
_D = 128
_HD = 64
_E = 320000
_NC = 2
_NS = 16
_NPAD = 10112
_RPS = _NPAD // _NS
_RPSL = _N - (_NS - 1) * _RPS
_CHUNK = 128
_CPT = 160
_EPAD = _NS * _CPT * _CHUNK
_NBUF = 2

_mesh = plsc.VectorSubcoreMesh(core_axis_name="c", subcore_axis_name="s")


@functools.partial(
    pl.kernel,
    mesh=_mesh,
    compiler_params=pltpu.CompilerParams(use_tc_tiling_on_sc=False),
    out_type=jax.ShapeDtypeStruct((_N, _D), jnp.float32),
    scratch_types=[
        pltpu.VMEM((_CPT // 2, _CHUNK), jnp.int32),
        pltpu.VMEM((_NBUF, _CHUNK), jnp.int32),
        pltpu.VMEM((_NBUF, _CHUNK), jnp.int32),
        pltpu.VMEM((_NBUF, _CHUNK, _HD), jnp.float32),
        pltpu.VMEM_SHARED((_NPAD, _HD), jnp.float32),
        pltpu.VMEM_SHARED((_NPAD, _HD), jnp.float32),
    ] + [pltpu.SemaphoreType.DMA] * _NBUF,
)
def _sc_aggregate(x_hbm, combo_hbm, out_hbm,
                  combo_v, sidx_v, didx_v, rows_v, x_sh, agg_sh, *sems):
    cid = lax.axis_index("c")
    sid = lax.axis_index("s")
    r0 = pl.multiple_of(sid * _RPS, 8)
    c0 = pl.multiple_of(cid * _HD, _HD)

    @pl.when(sid < _NS - 1)
    def _():
        pltpu.sync_copy(x_hbm.at[pl.ds(r0, _RPS), pl.ds(c0, _HD)],
                        x_sh.at[pl.ds(r0, _RPS)])
        pltpu.sync_copy(x_hbm.at[pl.ds(r0, _RPS), pl.ds(c0, _HD)],
                        agg_sh.at[pl.ds(r0, _RPS)])

    @pl.when(sid == _NS - 1)
    def _():
        pltpu.sync_copy(x_hbm.at[pl.ds(r0, _RPSL), pl.ds(c0, _HD)],
                        x_sh.at[pl.ds(r0, _RPSL)])
        pltpu.sync_copy(x_hbm.at[pl.ds(r0, _RPSL), pl.ds(c0, _HD)],
                        agg_sh.at[pl.ds(r0, _RPSL)])

    plsc.subcore_barrier()

    def unpack_idx(j, p):
        for k in range(_CHUNK // 16):
            c = combo_v[j, pl.ds(k * 16, 16)]
            sidx_v[p, pl.ds(k * 16, 16)] = lax.shift_right_logical(c, 14)
            didx_v[p, pl.ds(k * 16, 16)] = lax.bitwise_and(c, 16383)

    half = _CPT // 2
    for h in range(2):
        pltpu.sync_copy(combo_hbm.at[sid, pl.ds(h * half, half)], combo_v)
        for p in range(_NBUF):
            unpack_idx(p, p)
            pltpu.async_copy(x_sh.at[sidx_v.at[p]], rows_v.at[p], sems[p])

        @pl.loop(0, half, step=_NBUF)
        def _(j0):
            for p in range(_NBUF):
                j = j0 + p
                pltpu.make_async_copy(x_sh.at[sidx_v.at[p]], rows_v.at[p],
                                      sems[p]).wait()
                pltpu.sync_copy(rows_v.at[p], agg_sh.at[didx_v.at[p]],
                                add=True)

                @pl.when(j + _NBUF < half)
                def _():
                    unpack_idx(j + _NBUF, p)
                    pltpu.async_copy(x_sh.at[sidx_v.at[p]], rows_v.at[p],
                                     sems[p])

    plsc.subcore_barrier()

    @pl.when(sid < _NS - 1)
    def _():
        pltpu.sync_copy(agg_sh.at[pl.ds(r0, _RPS)],
                        out_hbm.at[pl.ds(r0, _RPS), pl.ds(c0, _HD)])

    @pl.when(sid == _NS - 1)
    def _():
        pltpu.sync_copy(agg_sh.at[pl.ds(r0, _RPSL)],
                        out_hbm.at[pl.ds(r0, _RPSL), pl.ds(c0, _HD)])


_EROW = _E // 128
_OROW = _EPAD // 128
_PBLK = _OROW // 8


def _tc_pack(srcdst):
    def body(x_ref, o_ref):
        row = lax.broadcasted_iota(jnp.int32, (_OROW, 128), 0)
        packed = jnp.left_shift(x_ref[0], 14) | x_ref[1]
        o_ref[...] = jnp.where(row < _EROW, packed, _N).reshape(
            _NS, _CPT, _CHUNK)

    return pl.pallas_call(
        body,
        out_shape=jax.ShapeDtypeStruct((_NS, _CPT, _CHUNK), jnp.int32),
    )(srcdst)


def _tc_tail(parts, w1, b1, w2, b2, gamma, beta):
    def body(p_ref, w1_ref, b1_ref, w2_ref, b2_ref, g_ref, bt_ref, o_ref):
        h = p_ref[...]
        h = jnp.dot(h, w1_ref[...], preferred_element_type=jnp.float32)
        h = jnp.maximum(h + b1_ref[...], 0.0)
        h = jnp.dot(h, w2_ref[...], preferred_element_type=jnp.float32)
        h = h + b2_ref[...]
        mean = jnp.mean(h, axis=0, keepdims=True)
        var = jnp.mean(h * h, axis=0, keepdims=True) - mean * mean
        scale = lax.rsqrt(var + _BN_EPS) * g_ref[...]
        o_ref[...] = jnp.maximum((h - mean) * scale + bt_ref[...], 0.0)

    return pl.pallas_call(
        body,
        out_shape=jax.ShapeDtypeStruct((_N, _D), jnp.float32),
    )(parts, w1, b1.reshape(1, _D), w2, b2.reshape(1, _D),
      gamma.reshape(1, _D), beta.reshape(1, _D))


def kernel(x, edge_index, W1, b1, W2, b2, gamma, beta):
    ei = edge_index.astype(jnp.int32)
    ei = jnp.pad(ei, ((0, 0), (0, _EPAD - _E)))
    combo = _tc_pack(ei.reshape(2, _OROW, 128))
    parts = _sc_aggregate(x, combo)
    return _tc_tail(parts, W1, b1, W2, b2, gamma, beta)
